# Initial kernel scaffold; baseline (speedup 1.0000x reference)
import jax
import jax.numpy as jnp
from jax.experimental import pallas as pl


def kernel(x, edge_index, edge_weight, W1, b1, W2, b2):
    raise NotImplementedError("write your pallas kernel here")


# speedup vs baseline: 36.2226x; 36.2226x over previous
import functools

import jax
import jax.numpy as jnp
from jax import lax
from jax.experimental import pallas as pl
from jax.experimental.pallas import tpu as pltpu
from jax.experimental.pallas import tpu_sc as plsc

N = 10000
E = 320000
HID = 16
NC = 2
NS = 16
NW = NC * NS
CH = 128
NCH = 80
EPT = CH * NCH
EP = EPT * NW
NP = 10240
RPT = NP // NS

_MESH = plsc.VectorSubcoreMesh(core_axis_name="c", subcore_axis_name="s")


@functools.partial(
    pl.kernel, mesh=_MESH,
    compiler_params=pltpu.CompilerParams(use_tc_tiling_on_sc=False),
    out_type=jax.ShapeDtypeStruct((NC, NP), jnp.float32),
    scratch_types=[
        pltpu.VMEM((NCH, CH), jnp.int32),
        pltpu.VMEM((NCH, CH), jnp.float32),
        pltpu.VMEM((RPT,), jnp.float32),
        pltpu.VMEM_SHARED((NP,), jnp.float32),
    ],
)
def _sc_degree(dst_hbm, ew_hbm, out_hbm, dst_v, ew_v, zbuf, acc):
    c = lax.axis_index("c")
    s = lax.axis_index("s")
    wid = s * NC + c
    pltpu.sync_copy(dst_hbm.at[wid], dst_v)
    pltpu.sync_copy(ew_hbm.at[wid], ew_v)

    def zb(i, carry):
        zbuf[pl.ds(i * 16, 16)] = jnp.zeros((16,), jnp.float32)
        return carry
    lax.fori_loop(0, RPT // 16, zb, None)
    pltpu.sync_copy(zbuf, acc.at[pl.ds(s * RPT, RPT)])
    plsc.subcore_barrier()

    def chunk(i, carry):
        pltpu.sync_copy(ew_v.at[i], acc.at[dst_v.at[i]], add=True)
        return carry
    lax.fori_loop(0, NCH, chunk, None)
    plsc.subcore_barrier()
    pltpu.sync_copy(acc.at[pl.ds(s * RPT, RPT)],
                    out_hbm.at[c, pl.ds(s * RPT, RPT)])


@functools.partial(
    pl.kernel, mesh=_MESH,
    compiler_params=pltpu.CompilerParams(use_tc_tiling_on_sc=False),
    out_type=jax.ShapeDtypeStruct((NC, NP, HID), jnp.float32),
    scratch_types=[
        pltpu.VMEM((NCH, CH), jnp.int32),
        pltpu.VMEM((NCH, CH), jnp.int32),
        pltpu.VMEM((NCH, CH), jnp.float32),
        pltpu.VMEM((CH, HID), jnp.float32),
        pltpu.VMEM((CH, HID), jnp.float32),
        pltpu.VMEM((RPT, HID), jnp.float32),
        pltpu.VMEM_SHARED((NP, HID), jnp.float32),
        pltpu.SemaphoreType.DMA,
        pltpu.SemaphoreType.DMA,
    ],
)
def _sc_aggregate(h_hbm, src_hbm, dst_hbm, ew_hbm, out_hbm,
                  src_v, dst_v, ew_v, r0, r1, zbuf, acc, sem0, sem1):
    c = lax.axis_index("c")
    s = lax.axis_index("s")
    wid = s * NC + c
    pltpu.sync_copy(src_hbm.at[wid], src_v)
    pltpu.sync_copy(dst_hbm.at[wid], dst_v)
    pltpu.sync_copy(ew_hbm.at[wid], ew_v)

    def zb(i, carry):
        zbuf[i, :] = jnp.zeros((HID,), jnp.float32)
        return carry
    lax.fori_loop(0, RPT, zb, None)
    pltpu.sync_copy(zbuf, acc.at[pl.ds(s * RPT, RPT)])
    plsc.subcore_barrier()

    def scale(rbuf, ci):
        def body(g, carry):
            w = ew_v[ci, pl.ds(g * 16, 16)]
            base = g * 16
            for j in range(16):
                rbuf[base + j, :] = rbuf[base + j, :] * w[j]
            return carry
        lax.fori_loop(0, CH // 16, body, None)

    pltpu.async_copy(h_hbm.at[src_v.at[0]], r0, sem0)

    def pair(i, carry):
        c0 = 2 * i
        pltpu.async_copy(h_hbm.at[src_v.at[c0 + 1]], r1, sem1)
        pltpu.make_async_copy(h_hbm.at[src_v.at[c0]], r0, sem0).wait()
        scale(r0, c0)
        pltpu.sync_copy(r0, acc.at[dst_v.at[c0]], add=True)

        @pl.when(i < NCH // 2 - 1)
        def _prefetch():
            pltpu.async_copy(h_hbm.at[src_v.at[c0 + 2]], r0, sem0)

        pltpu.make_async_copy(h_hbm.at[src_v.at[c0 + 1]], r1, sem1).wait()
        scale(r1, c0 + 1)
        pltpu.sync_copy(r1, acc.at[dst_v.at[c0 + 1]], add=True)
        return carry
    lax.fori_loop(0, NCH // 2, pair, None)
    plsc.subcore_barrier()
    pltpu.sync_copy(acc.at[pl.ds(s * RPT, RPT)],
                    out_hbm.at[c, pl.ds(s * RPT, RPT)])


def _tc_stage1(x, W1, deg_p):
    def body(x_ref, w_ref, degp_ref, h1p_ref, dinv_ref):
        deg = degp_ref[0, :] + degp_ref[1, :] + 1.0
        dinv = jnp.where(deg > 0, lax.rsqrt(jnp.maximum(deg, 1e-12)), 0.0)
        dinv_ref[...] = dinv[:, None]
        h = jnp.dot(x_ref[...], w_ref[...], preferred_element_type=jnp.float32)
        h1p_ref[...] = h * dinv[:, None]
    return pl.pallas_call(
        body,
        out_shape=(jax.ShapeDtypeStruct((N, HID), jnp.float32),
                   jax.ShapeDtypeStruct((N, 1), jnp.float32)),
    )(x, W1, deg_p)


def _tc_stage2(agg1, h1p, dinv, b1):
    def body(agg_ref, h1p_ref, dinv_ref, b1_ref, zp_ref):
        dv = dinv_ref[...]
        out1 = dv * (agg_ref[0] + agg_ref[1] + h1p_ref[...]) + b1_ref[...]
        zp_ref[...] = jnp.maximum(out1, 0.0) * dv
    return pl.pallas_call(
        body,
        out_shape=jax.ShapeDtypeStruct((N, HID), jnp.float32),
    )(agg1, h1p, dinv, b1)


def _tc_stage3(agg2, zp, dinv, W2, b2):
    def body(agg_ref, zp_ref, dinv_ref, w_ref, b_ref, out_ref):
        a = (agg_ref[0] + agg_ref[1] + zp_ref[...]) * dinv_ref[...]
        out_ref[...] = (
            jnp.dot(a, w_ref[...], preferred_element_type=jnp.float32)
            + b_ref[...])
    return pl.pallas_call(
        body,
        out_shape=jax.ShapeDtypeStruct((N, W2.shape[1]), jnp.float32),
    )(agg2, zp, dinv, W2, b2)


def kernel(x, edge_index, edge_weight, W1, b1, W2, b2):
    src = edge_index[0]
    dst = edge_index[1]
    pad = EP - E
    src_t = jnp.pad(src, (0, pad)).reshape(NW, NCH, CH)
    dst_t = jnp.pad(dst, (0, pad)).reshape(NW, NCH, CH)
    ew_t = jnp.pad(edge_weight, (0, pad)).reshape(NW, NCH, CH)

    deg_p = _sc_degree(dst_t, ew_t)[:, :N]
    h1p, dinv = _tc_stage1(x, W1, deg_p)
    agg1 = _sc_aggregate(h1p, src_t, dst_t, ew_t)[:, :N, :]
    zp = _tc_stage2(agg1, h1p, dinv, b1)
    agg2 = _sc_aggregate(zp, src_t, dst_t, ew_t)[:, :N, :]
    return _tc_stage3(agg2, zp, dinv, W2, b2)


# speedup vs baseline: 39.7024x; 1.0961x over previous
import functools

import jax
import jax.numpy as jnp
from jax import lax
from jax.experimental import pallas as pl
from jax.experimental.pallas import tpu as pltpu
from jax.experimental.pallas import tpu_sc as plsc

N = 10000
E = 320000
HID = 16
NC = 2
NS = 16
NW = NC * NS
CH = 128
NCH = 80
EPT = CH * NCH
EP = EPT * NW
NP = 10240
RPT = NP // NS

_MESH = plsc.VectorSubcoreMesh(core_axis_name="c", subcore_axis_name="s")


@functools.partial(
    pl.kernel, mesh=_MESH,
    compiler_params=pltpu.CompilerParams(use_tc_tiling_on_sc=False),
    out_type=jax.ShapeDtypeStruct((NC, NP), jnp.float32),
    scratch_types=[
        pltpu.VMEM((NCH, CH), jnp.int32),
        pltpu.VMEM((NCH, CH), jnp.float32),
        pltpu.VMEM((RPT,), jnp.float32),
        pltpu.VMEM_SHARED((NP,), jnp.float32),
    ],
)
def _sc_degree(dst_hbm, ew_hbm, out_hbm, dst_v, ew_v, zbuf, acc):
    c = lax.axis_index("c")
    s = lax.axis_index("s")
    wid = s * NC + c
    pltpu.sync_copy(dst_hbm.at[wid], dst_v)
    pltpu.sync_copy(ew_hbm.at[wid], ew_v)

    def zb(i, carry):
        zbuf[pl.ds(i * 16, 16)] = jnp.zeros((16,), jnp.float32)
        return carry
    lax.fori_loop(0, RPT // 16, zb, None)
    pltpu.sync_copy(zbuf, acc.at[pl.ds(s * RPT, RPT)])
    plsc.subcore_barrier()

    def chunk(i, carry):
        pltpu.sync_copy(ew_v.at[i], acc.at[dst_v.at[i]], add=True)
        return carry
    lax.fori_loop(0, NCH, chunk, None)
    plsc.subcore_barrier()
    pltpu.sync_copy(acc.at[pl.ds(s * RPT, RPT)],
                    out_hbm.at[c, pl.ds(s * RPT, RPT)])


@functools.partial(
    pl.kernel, mesh=_MESH,
    compiler_params=pltpu.CompilerParams(use_tc_tiling_on_sc=False),
    out_type=jax.ShapeDtypeStruct((NC, NP, HID), jnp.float32),
    scratch_types=[
        pltpu.VMEM((NCH, CH), jnp.int32),
        pltpu.VMEM((NCH, CH), jnp.int32),
        pltpu.VMEM((NCH, CH), jnp.float32),
        pltpu.VMEM((CH, HID), jnp.float32),
        pltpu.VMEM((CH, HID), jnp.float32),
        pltpu.VMEM((CH, HID), jnp.float32),
        pltpu.VMEM((CH, HID), jnp.float32),
        pltpu.VMEM((RPT, HID), jnp.float32),
        pltpu.VMEM_SHARED((NP, HID), jnp.float32),
        pltpu.SemaphoreType.DMA,
        pltpu.SemaphoreType.DMA,
        pltpu.SemaphoreType.DMA,
        pltpu.SemaphoreType.DMA,
        pltpu.SemaphoreType.DMA,
        pltpu.SemaphoreType.DMA,
        pltpu.SemaphoreType.DMA,
        pltpu.SemaphoreType.DMA,
    ],
)
def _sc_aggregate(h_hbm, src_hbm, dst_hbm, ew_hbm, out_hbm,
                  src_v, dst_v, ew_v, r0, r1, r2, r3, zbuf, acc,
                  g0, g1, g2, g3, s0, s1, s2, s3):
    c = lax.axis_index("c")
    s = lax.axis_index("s")
    wid = s * NC + c
    pltpu.sync_copy(src_hbm.at[wid], src_v)
    pltpu.sync_copy(dst_hbm.at[wid], dst_v)
    pltpu.sync_copy(ew_hbm.at[wid], ew_v)
    rbufs = (r0, r1, r2, r3)
    gsems = (g0, g1, g2, g3)
    ssems = (s0, s1, s2, s3)

    def zb(i, carry):
        zbuf[i, :] = jnp.zeros((HID,), jnp.float32)
        return carry
    lax.fori_loop(0, RPT, zb, None)
    pltpu.sync_copy(zbuf, acc.at[pl.ds(s * RPT, RPT)])
    plsc.subcore_barrier()

    def scale(rbuf, ci):
        def body(g, carry):
            w = ew_v[ci, pl.ds(g * 16, 16)]
            base = g * 16
            for j in range(16):
                rbuf[base + j, :] = rbuf[base + j, :] * w[j]
            return carry
        lax.fori_loop(0, CH // 16, body, None)

    pltpu.async_copy(h_hbm.at[src_v.at[0]], r0, g0)
    pltpu.async_copy(h_hbm.at[src_v.at[1]], r1, g1)

    def quad(i, carry):
        for b in range(4):
            ci = 4 * i + b
            pltpu.make_async_copy(h_hbm.at[src_v.at[ci]], rbufs[b], gsems[b]).wait()
            scale(rbufs[b], ci)
            pltpu.async_copy(rbufs[b], acc.at[dst_v.at[ci]], ssems[b], add=True)
            bp = (b + 2) % 4
            @pl.when(ci >= 2)
            def _drain():
                pltpu.make_async_copy(
                    rbufs[bp], acc.at[dst_v.at[ci - 2]], ssems[bp]).wait()
            @pl.when(ci + 2 < NCH)
            def _prefetch():
                pltpu.async_copy(
                    h_hbm.at[src_v.at[ci + 2]], rbufs[bp], gsems[bp])
        return carry
    lax.fori_loop(0, NCH // 4, quad, None)
    pltpu.make_async_copy(r2, acc.at[dst_v.at[NCH - 2]], s2).wait()
    pltpu.make_async_copy(r3, acc.at[dst_v.at[NCH - 1]], s3).wait()
    plsc.subcore_barrier()
    pltpu.sync_copy(acc.at[pl.ds(s * RPT, RPT)],
                    out_hbm.at[c, pl.ds(s * RPT, RPT)])


def _tc_stage1(x, W1, deg_p):
    def body(x_ref, w_ref, degp_ref, h1p_ref, dinv_ref):
        deg = degp_ref[0, :N] + degp_ref[1, :N] + 1.0
        dinv = jnp.where(deg > 0, lax.rsqrt(jnp.maximum(deg, 1e-12)), 0.0)
        dinv_ref[...] = dinv[:, None]
        h = jnp.dot(x_ref[...], w_ref[...], preferred_element_type=jnp.float32)
        h1p_ref[...] = h * dinv[:, None]
    return pl.pallas_call(
        body,
        out_shape=(jax.ShapeDtypeStruct((N, HID), jnp.float32),
                   jax.ShapeDtypeStruct((N, 1), jnp.float32)),
    )(x, W1, deg_p)


def _tc_stage2(agg1, h1p, dinv, b1):
    def body(agg_ref, h1p_ref, dinv_ref, b1_ref, zp_ref):
        dv = dinv_ref[...]
        out1 = dv * (agg_ref[0, :N] + agg_ref[1, :N] + h1p_ref[...]) + b1_ref[...]
        zp_ref[...] = jnp.maximum(out1, 0.0) * dv
    return pl.pallas_call(
        body,
        out_shape=jax.ShapeDtypeStruct((N, HID), jnp.float32),
    )(agg1, h1p, dinv, b1)


def _tc_stage3(agg2, zp, dinv, W2, b2):
    def body(agg_ref, zp_ref, dinv_ref, w_ref, b_ref, out_ref):
        a = (agg_ref[0, :N] + agg_ref[1, :N] + zp_ref[...]) * dinv_ref[...]
        out_ref[...] = (
            jnp.dot(a, w_ref[...], preferred_element_type=jnp.float32)
            + b_ref[...])
    return pl.pallas_call(
        body,
        out_shape=jax.ShapeDtypeStruct((N, W2.shape[1]), jnp.float32),
    )(agg2, zp, dinv, W2, b2)


def kernel(x, edge_index, edge_weight, W1, b1, W2, b2):
    src = edge_index[0]
    dst = edge_index[1]
    pad = EP - E
    src_t = jnp.pad(src, (0, pad)).reshape(NW, NCH, CH)
    dst_t = jnp.pad(dst, (0, pad)).reshape(NW, NCH, CH)
    ew_t = jnp.pad(edge_weight, (0, pad)).reshape(NW, NCH, CH)

    deg_p = _sc_degree(dst_t, ew_t)
    h1p, dinv = _tc_stage1(x, W1, deg_p)
    agg1 = _sc_aggregate(h1p, src_t, dst_t, ew_t)
    zp = _tc_stage2(agg1, h1p, dinv, b1)
    agg2 = _sc_aggregate(zp, src_t, dst_t, ew_t)
    return _tc_stage3(agg2, zp, dinv, W2, b2)


# speedup vs baseline: 45.9334x; 1.1569x over previous
import functools

import jax
import jax.numpy as jnp
from jax import lax
from jax.experimental import pallas as pl
from jax.experimental.pallas import tpu as pltpu
from jax.experimental.pallas import tpu_sc as plsc

N = 10000
E = 320000
HID = 16
NC = 2
NS = 16
CH = 128
NCH0 = 96
NCH1 = 64
TOTCH = NS * (NCH0 + NCH1)
C0TOT = NS * NCH0
EP = TOTCH * CH
NP = 10240
RPT = NP // NS

_MESH = plsc.VectorSubcoreMesh(core_axis_name="c", subcore_axis_name="s")


@functools.partial(
    pl.kernel, mesh=_MESH,
    compiler_params=pltpu.CompilerParams(use_tc_tiling_on_sc=False),
    out_type=jax.ShapeDtypeStruct((NC, NP), jnp.float32),
    scratch_types=[
        pltpu.VMEM((NCH0, CH), jnp.int32),
        pltpu.VMEM((NCH0, CH), jnp.float32),
        pltpu.VMEM((RPT,), jnp.float32),
        pltpu.VMEM_SHARED((NP,), jnp.float32),
    ],
)
def _sc_degree(ei_hbm, ew_hbm, out_hbm, dst_v, ew_v, zbuf, acc):
    c = lax.axis_index("c")
    s = lax.axis_index("s")

    def zb(i, carry):
        zbuf[pl.ds(i * 16, 16)] = jnp.zeros((16,), jnp.float32)
        return carry
    lax.fori_loop(0, RPT // 16, zb, None)
    pltpu.sync_copy(zbuf, acc.at[pl.ds(s * RPT, RPT)])
    plsc.subcore_barrier()

    def run(nch, base):
        pltpu.sync_copy(ei_hbm.at[1, pl.ds(base, nch)], dst_v.at[pl.ds(0, nch)])
        pltpu.sync_copy(ew_hbm.at[pl.ds(base, nch)], ew_v.at[pl.ds(0, nch)])

        def chunk(i, carry):
            pltpu.sync_copy(ew_v.at[i], acc.at[dst_v.at[i]], add=True)
            return carry
        lax.fori_loop(0, nch, chunk, None)

    @pl.when(c == 0)
    def _c0():
        run(NCH0, s * NCH0)

    @pl.when(c == 1)
    def _c1():
        run(NCH1, C0TOT + s * NCH1)

    plsc.subcore_barrier()
    pltpu.sync_copy(acc.at[pl.ds(s * RPT, RPT)],
                    out_hbm.at[c, pl.ds(s * RPT, RPT)])


@functools.partial(
    pl.kernel, mesh=_MESH,
    compiler_params=pltpu.CompilerParams(use_tc_tiling_on_sc=False),
    out_type=jax.ShapeDtypeStruct((NC, NP, HID), jnp.float32),
    scratch_types=[
        pltpu.VMEM((NCH0, CH), jnp.int32),
        pltpu.VMEM((NCH0, CH), jnp.int32),
        pltpu.VMEM((NCH0, CH), jnp.float32),
        pltpu.VMEM((CH, HID), jnp.float32),
        pltpu.VMEM((CH, HID), jnp.float32),
        pltpu.VMEM((CH, HID), jnp.float32),
        pltpu.VMEM((CH, HID), jnp.float32),
        pltpu.VMEM((RPT, HID), jnp.float32),
        pltpu.VMEM_SHARED((NP, HID), jnp.float32),
        pltpu.SemaphoreType.DMA,
        pltpu.SemaphoreType.DMA,
        pltpu.SemaphoreType.DMA,
        pltpu.SemaphoreType.DMA,
        pltpu.SemaphoreType.DMA,
        pltpu.SemaphoreType.DMA,
        pltpu.SemaphoreType.DMA,
        pltpu.SemaphoreType.DMA,
    ],
)
def _sc_aggregate(h_hbm, ei_hbm, ew_hbm, out_hbm,
                  src_v, dst_v, ew_v, r0, r1, r2, r3, zbuf, acc,
                  g0, g1, g2, g3, s0, s1, s2, s3):
    c = lax.axis_index("c")
    s = lax.axis_index("s")
    rbufs = (r0, r1, r2, r3)
    gsems = (g0, g1, g2, g3)
    ssems = (s0, s1, s2, s3)

    def zb(i, carry):
        zbuf[i, :] = jnp.zeros((HID,), jnp.float32)
        return carry
    lax.fori_loop(0, RPT, zb, None)
    pltpu.sync_copy(zbuf, acc.at[pl.ds(s * RPT, RPT)])
    plsc.subcore_barrier()

    def scale(rbuf, ci):
        def body(g, carry):
            w = ew_v[ci, pl.ds(g * 16, 16)]
            base = g * 16
            for j in range(16):
                rbuf[base + j, :] = rbuf[base + j, :] * w[j]
            return carry
        lax.fori_loop(0, CH // 16, body, None)

    def run(nch, base):
        pltpu.sync_copy(ei_hbm.at[0, pl.ds(base, nch)], src_v.at[pl.ds(0, nch)])
        pltpu.sync_copy(ei_hbm.at[1, pl.ds(base, nch)], dst_v.at[pl.ds(0, nch)])
        pltpu.sync_copy(ew_hbm.at[pl.ds(base, nch)], ew_v.at[pl.ds(0, nch)])

        pltpu.async_copy(h_hbm.at[src_v.at[0]], r0, g0)
        pltpu.async_copy(h_hbm.at[src_v.at[1]], r1, g1)

        def quad(i, carry):
            for b in range(4):
                ci = 4 * i + b
                pltpu.make_async_copy(
                    h_hbm.at[src_v.at[ci]], rbufs[b], gsems[b]).wait()
                scale(rbufs[b], ci)
                pltpu.async_copy(rbufs[b], acc.at[dst_v.at[ci]], ssems[b],
                                 add=True)
                bp = (b + 2) % 4

                @pl.when(ci >= 2)
                def _drain():
                    pltpu.make_async_copy(
                        rbufs[bp], acc.at[dst_v.at[ci - 2]], ssems[bp]).wait()

                @pl.when(ci + 2 < nch)
                def _prefetch():
                    pltpu.async_copy(
                        h_hbm.at[src_v.at[ci + 2]], rbufs[bp], gsems[bp])
            return carry
        lax.fori_loop(0, nch // 4, quad, None)
        pltpu.make_async_copy(r2, acc.at[dst_v.at[nch - 2]], s2).wait()
        pltpu.make_async_copy(r3, acc.at[dst_v.at[nch - 1]], s3).wait()

    @pl.when(c == 0)
    def _c0():
        run(NCH0, s * NCH0)

    @pl.when(c == 1)
    def _c1():
        run(NCH1, C0TOT + s * NCH1)

    plsc.subcore_barrier()
    pltpu.sync_copy(acc.at[pl.ds(s * RPT, RPT)],
                    out_hbm.at[c, pl.ds(s * RPT, RPT)])


def _tc_stage1(x, W1, deg_p):
    def body(x_ref, w_ref, degp_ref, h1p_ref, dinv_ref):
        deg = degp_ref[0, :N] + degp_ref[1, :N] + 1.0
        dinv = jnp.where(deg > 0, lax.rsqrt(jnp.maximum(deg, 1e-12)), 0.0)
        dinv_ref[...] = dinv[:, None]
        h = jnp.dot(x_ref[...], w_ref[...], preferred_element_type=jnp.float32)
        h1p_ref[...] = h * dinv[:, None]
    return pl.pallas_call(
        body,
        out_shape=(jax.ShapeDtypeStruct((N, HID), jnp.float32),
                   jax.ShapeDtypeStruct((N, 1), jnp.float32)),
    )(x, W1, deg_p)


def _tc_stage2(agg1, h1p, dinv, b1):
    def body(agg_ref, h1p_ref, dinv_ref, b1_ref, zp_ref):
        dv = dinv_ref[...]
        out1 = dv * (agg_ref[0, :N] + agg_ref[1, :N] + h1p_ref[...]) + b1_ref[...]
        zp_ref[...] = jnp.maximum(out1, 0.0) * dv
    return pl.pallas_call(
        body,
        out_shape=jax.ShapeDtypeStruct((N, HID), jnp.float32),
    )(agg1, h1p, dinv, b1)


def _tc_stage3(agg2, zp, dinv, W2, b2):
    def body(agg_ref, zp_ref, dinv_ref, w_ref, b_ref, out_ref):
        a = (agg_ref[0, :N] + agg_ref[1, :N] + zp_ref[...]) * dinv_ref[...]
        out_ref[...] = (
            jnp.dot(a, w_ref[...], preferred_element_type=jnp.float32)
            + b_ref[...])
    return pl.pallas_call(
        body,
        out_shape=jax.ShapeDtypeStruct((N, W2.shape[1]), jnp.float32),
    )(agg2, zp, dinv, W2, b2)


def kernel(x, edge_index, edge_weight, W1, b1, W2, b2):
    pad = EP - E
    ei_t = jnp.pad(edge_index, ((0, 0), (0, pad))).reshape(2, TOTCH, CH)
    ew_t = jnp.pad(edge_weight, (0, pad)).reshape(TOTCH, CH)

    deg_p = _sc_degree(ei_t, ew_t)
    h1p, dinv = _tc_stage1(x, W1, deg_p)
    agg1 = _sc_aggregate(h1p, ei_t, ew_t)
    zp = _tc_stage2(agg1, h1p, dinv, b1)
    agg2 = _sc_aggregate(zp, ei_t, ew_t)
    return _tc_stage3(agg2, zp, dinv, W2, b2)


# speedup vs baseline: 48.9972x; 1.0667x over previous
import functools

import jax
import jax.numpy as jnp
from jax import lax
from jax.experimental import pallas as pl
from jax.experimental.pallas import tpu as pltpu
from jax.experimental.pallas import tpu_sc as plsc

N = 10000
E = 320000
HID = 16
NC = 2
NS = 16
CH = 128
NCH0 = 96
NCH1 = 64
TOTCH = NS * (NCH0 + NCH1)
C0TOT = NS * NCH0
EP = TOTCH * CH
NP = 10240
RPT = NP // NS

_MESH = plsc.VectorSubcoreMesh(core_axis_name="c", subcore_axis_name="s")


@functools.partial(
    pl.kernel, mesh=_MESH,
    compiler_params=pltpu.CompilerParams(use_tc_tiling_on_sc=False),
    out_type=jax.ShapeDtypeStruct((NC, NP), jnp.float32),
    scratch_types=[
        pltpu.VMEM((NCH0, CH), jnp.int32),
        pltpu.VMEM((NCH0, CH), jnp.float32),
        pltpu.VMEM((RPT,), jnp.float32),
        pltpu.VMEM_SHARED((NP,), jnp.float32),
    ],
)
def _sc_degree(ei_hbm, ew_hbm, out_hbm, dst_v, ew_v, zbuf, acc):
    c = lax.axis_index("c")
    s = lax.axis_index("s")

    def zb(i, carry):
        zbuf[pl.ds(i * 16, 16)] = jnp.zeros((16,), jnp.float32)
        return carry
    lax.fori_loop(0, RPT // 16, zb, None)
    pltpu.sync_copy(zbuf, acc.at[pl.ds(s * RPT, RPT)])
    plsc.subcore_barrier()

    def run(nch, base):
        pltpu.sync_copy(ei_hbm.at[1, pl.ds(base, nch)], dst_v.at[pl.ds(0, nch)])
        pltpu.sync_copy(ew_hbm.at[pl.ds(base, nch)], ew_v.at[pl.ds(0, nch)])

        def chunk(i, carry):
            pltpu.sync_copy(ew_v.at[i], acc.at[dst_v.at[i]], add=True)
            return carry
        lax.fori_loop(0, nch, chunk, None)

    @pl.when(c == 0)
    def _c0():
        run(NCH0, s * NCH0)

    @pl.when(c == 1)
    def _c1():
        run(NCH1, C0TOT + s * NCH1)

    plsc.subcore_barrier()
    pltpu.sync_copy(acc.at[pl.ds(s * RPT, RPT)],
                    out_hbm.at[c, pl.ds(s * RPT, RPT)])


@functools.partial(
    pl.kernel, mesh=_MESH,
    compiler_params=pltpu.CompilerParams(use_tc_tiling_on_sc=False),
    out_type=jax.ShapeDtypeStruct((NC, NP, HID), jnp.float32),
    scratch_types=[
        pltpu.VMEM((NCH0, CH), jnp.int32),
        pltpu.VMEM((NCH0, CH), jnp.int32),
        pltpu.VMEM((NCH0, CH), jnp.float32),
        pltpu.VMEM((CH, HID), jnp.float32),
        pltpu.VMEM((CH, HID), jnp.float32),
        pltpu.VMEM((CH, HID), jnp.float32),
        pltpu.VMEM((CH, HID), jnp.float32),
        pltpu.VMEM((RPT, HID), jnp.float32),
        pltpu.VMEM_SHARED((NP, HID), jnp.float32),
        pltpu.SemaphoreType.DMA,
        pltpu.SemaphoreType.DMA,
        pltpu.SemaphoreType.DMA,
        pltpu.SemaphoreType.DMA,
        pltpu.SemaphoreType.DMA,
        pltpu.SemaphoreType.DMA,
        pltpu.SemaphoreType.DMA,
        pltpu.SemaphoreType.DMA,
    ],
)
def _sc_aggregate(h_hbm, ei_hbm, ew_hbm, out_hbm,
                  src_v, dst_v, ew_v, r0, r1, r2, r3, zbuf, acc,
                  g0, g1, g2, g3, s0, s1, s2, s3):
    c = lax.axis_index("c")
    s = lax.axis_index("s")
    rbufs = (r0, r1, r2, r3)
    gsems = (g0, g1, g2, g3)
    ssems = (s0, s1, s2, s3)

    def zb(i, carry):
        zbuf[i, :] = jnp.zeros((HID,), jnp.float32)
        return carry
    lax.fori_loop(0, RPT, zb, None)
    pltpu.sync_copy(zbuf, acc.at[pl.ds(s * RPT, RPT)])
    plsc.subcore_barrier()

    def scale(rbuf, ci):
        def body(g, carry):
            w = ew_v[ci, pl.ds(g * 16, 16)]
            base = g * 16
            for j in range(16):
                rbuf[base + j, :] = rbuf[base + j, :] * w[j]
            return carry
        lax.fori_loop(0, CH // 16, body, None)

    def run(nch, base):
        pltpu.sync_copy(ei_hbm.at[0, pl.ds(base, nch)], src_v.at[pl.ds(0, nch)])
        pltpu.sync_copy(ei_hbm.at[1, pl.ds(base, nch)], dst_v.at[pl.ds(0, nch)])
        pltpu.sync_copy(ew_hbm.at[pl.ds(base, nch)], ew_v.at[pl.ds(0, nch)])

        pltpu.async_copy(h_hbm.at[src_v.at[0]], r0, g0)
        pltpu.async_copy(h_hbm.at[src_v.at[1]], r1, g1)

        def quad(i, carry):
            for b in range(4):
                ci = 4 * i + b
                pltpu.make_async_copy(
                    h_hbm.at[src_v.at[ci]], rbufs[b], gsems[b]).wait()
                scale(rbufs[b], ci)
                pltpu.async_copy(rbufs[b], acc.at[dst_v.at[ci]], ssems[b],
                                 add=True)
                bp = (b + 2) % 4

                @pl.when(ci >= 2)
                def _drain():
                    pltpu.make_async_copy(
                        rbufs[bp], acc.at[dst_v.at[ci - 2]], ssems[bp]).wait()

                @pl.when(ci + 2 < nch)
                def _prefetch():
                    pltpu.async_copy(
                        h_hbm.at[src_v.at[ci + 2]], rbufs[bp], gsems[bp])
            return carry
        lax.fori_loop(0, nch // 4, quad, None)
        pltpu.make_async_copy(r2, acc.at[dst_v.at[nch - 2]], s2).wait()
        pltpu.make_async_copy(r3, acc.at[dst_v.at[nch - 1]], s3).wait()

    @pl.when(c == 0)
    def _c0():
        run(NCH0, s * NCH0)

    @pl.when(c == 1)
    def _c1():
        run(NCH1, C0TOT + s * NCH1)

    plsc.subcore_barrier()
    pltpu.sync_copy(acc.at[pl.ds(s * RPT, RPT)],
                    out_hbm.at[c, pl.ds(s * RPT, RPT)])


RPW = NP // (NC * NS)


@functools.partial(
    pl.kernel, mesh=_MESH,
    compiler_params=pltpu.CompilerParams(use_tc_tiling_on_sc=False),
    out_type=jax.ShapeDtypeStruct((NP, HID), jnp.float32),
    scratch_types=[
        pltpu.VMEM((RPW, HID), jnp.float32),
        pltpu.VMEM((RPW, HID), jnp.float32),
        pltpu.VMEM((RPW, HID), jnp.float32),
        pltpu.VMEM((RPW,), jnp.float32),
        pltpu.VMEM((HID,), jnp.float32),
    ],
)
def _sc_relu_scale(agg_hbm, h1p_hbm, dinv_hbm, b1_hbm, out_hbm,
                   a0, a1, hp, dv, b1v):
    c = lax.axis_index("c")
    s = lax.axis_index("s")
    base = (s * NC + c) * RPW
    pltpu.sync_copy(agg_hbm.at[0, pl.ds(base, RPW)], a0)
    pltpu.sync_copy(agg_hbm.at[1, pl.ds(base, RPW)], a1)
    pltpu.sync_copy(h1p_hbm.at[pl.ds(base, RPW)], hp)
    pltpu.sync_copy(dinv_hbm.at[pl.ds(base, RPW)], dv)
    pltpu.sync_copy(b1_hbm, b1v)
    bvec = b1v[...]

    def group(g, carry):
        dvv = dv[pl.ds(g * 16, 16)]
        for j in range(16):
            n = g * 16 + j
            row = (a0[n, :] + a1[n, :] + hp[n, :]) * dvv[j] + bvec
            a0[n, :] = jnp.maximum(row, 0.0) * dvv[j]
        return carry
    lax.fori_loop(0, RPW // 16, group, None)
    pltpu.sync_copy(a0, out_hbm.at[pl.ds(base, RPW)])


@functools.partial(
    pl.kernel, mesh=_MESH,
    compiler_params=pltpu.CompilerParams(use_tc_tiling_on_sc=False),
    out_type=jax.ShapeDtypeStruct((NP, HID), jnp.float32),
    scratch_types=[
        pltpu.VMEM((RPW, HID), jnp.float32),
        pltpu.VMEM((RPW, HID), jnp.float32),
        pltpu.VMEM((RPW, HID), jnp.float32),
        pltpu.VMEM((RPW,), jnp.float32),
    ],
)
def _sc_sum_scale(agg_hbm, zp_hbm, dinv_hbm, out_hbm, a0, a1, zp, dv):
    c = lax.axis_index("c")
    s = lax.axis_index("s")
    base = (s * NC + c) * RPW
    pltpu.sync_copy(agg_hbm.at[0, pl.ds(base, RPW)], a0)
    pltpu.sync_copy(agg_hbm.at[1, pl.ds(base, RPW)], a1)
    pltpu.sync_copy(zp_hbm.at[pl.ds(base, RPW)], zp)
    pltpu.sync_copy(dinv_hbm.at[pl.ds(base, RPW)], dv)

    def group(g, carry):
        dvv = dv[pl.ds(g * 16, 16)]
        for j in range(16):
            n = g * 16 + j
            a0[n, :] = (a0[n, :] + a1[n, :] + zp[n, :]) * dvv[j]
        return carry
    lax.fori_loop(0, RPW // 16, group, None)
    pltpu.sync_copy(a0, out_hbm.at[pl.ds(base, RPW)])


def _tc_stage1(x, W1, deg_p):
    def body(x_ref, w_ref, degp_ref, h1p_ref, dinv_ref):
        deg = degp_ref[0, :] + degp_ref[1, :] + 1.0
        dinv = jnp.where(deg > 0, lax.rsqrt(jnp.maximum(deg, 1e-12)), 0.0)
        dinv_ref[...] = dinv
        h = jnp.dot(x_ref[...], w_ref[...], preferred_element_type=jnp.float32)
        h1p_ref[pl.ds(0, N), :] = h * dinv[:N, None]
        h1p_ref[pl.ds(N, NP - N), :] = jnp.zeros((NP - N, HID), jnp.float32)
    return pl.pallas_call(
        body,
        out_shape=(jax.ShapeDtypeStruct((NP, HID), jnp.float32),
                   jax.ShapeDtypeStruct((NP,), jnp.float32)),
    )(x, W1, deg_p)


def _tc_matmul2(sarr, W2, b2):
    def body(s_ref, w_ref, b_ref, out_ref):
        out_ref[...] = (
            jnp.dot(s_ref[0:N, :], w_ref[...],
                    preferred_element_type=jnp.float32)
            + b_ref[...])
    return pl.pallas_call(
        body,
        out_shape=jax.ShapeDtypeStruct((N, W2.shape[1]), jnp.float32),
    )(sarr, W2, b2)


def kernel(x, edge_index, edge_weight, W1, b1, W2, b2):
    pad = EP - E
    ei_t = jnp.pad(edge_index, ((0, 0), (0, pad))).reshape(2, TOTCH, CH)
    ew_t = jnp.pad(edge_weight, (0, pad)).reshape(TOTCH, CH)

    deg_p = _sc_degree(ei_t, ew_t)
    h1p, dinv = _tc_stage1(x, W1, deg_p)
    agg1 = _sc_aggregate(h1p, ei_t, ew_t)
    zp = _sc_relu_scale(agg1, h1p, dinv, b1)
    agg2 = _sc_aggregate(zp, ei_t, ew_t)
    sarr = _sc_sum_scale(agg2, zp, dinv)
    return _tc_matmul2(sarr, W2, b2)
